# Initial kernel scaffold; baseline (speedup 1.0000x reference)
import jax
import jax.numpy as jnp
from jax.experimental import pallas as pl


def kernel(x, adj, W, b, mlp_W, mlp_b):
    raise NotImplementedError("write your pallas kernel here")


# speedup vs baseline: 7.0512x; 7.0512x over previous
import functools

import jax
import jax.numpy as jnp
from jax import lax
from jax.experimental import pallas as pl
from jax.experimental.pallas import tpu as pltpu
from jax.experimental.pallas import tpu_sc as plsc

_C = 128
_NC = 2
_NS = 16


def _make_sc_agg(NP, H, E):
    nchunk = E // _C
    nw = _NC * _NS
    rps = NP // _NS
    mesh = plsc.VectorSubcoreMesh(
        core_axis_name="c", subcore_axis_name="s",
        num_cores=_NC, num_subcores=_NS)

    @functools.partial(
        pl.kernel,
        out_type=jax.ShapeDtypeStruct((_NC, NP, H), jnp.float32),
        mesh=mesh,
        scratch_types=[
            pltpu.VMEM((_C,), jnp.int32),
            pltpu.VMEM((_C,), jnp.int32),
            pltpu.VMEM((_C, H), jnp.float32),
            pltpu.VMEM_SHARED((NP, H), jnp.float32),
            pltpu.SemaphoreType.DMA,
        ],
    )
    def sc_agg(x_hbm, adj_hbm, out_hbm, src_v, dst_v, rows_v, accum, sem):
        c = lax.axis_index("c")
        s = lax.axis_index("s")
        wid = s * _NC + c
        base_row = s * rps

        def zbody(i, carry):
            for j in range(H // 16):
                rows_v[i, pl.ds(j * 16, 16)] = jnp.zeros((16,), jnp.float32)
            return carry

        lax.fori_loop(0, _C, zbody, 0)
        for k in range(rps // _C):
            pltpu.sync_copy(rows_v, accum.at[pl.ds(base_row + k * _C, _C)])
        plsc.subcore_barrier()

        nk = (nchunk - wid + nw - 1) // nw

        def body(k, carry):
            base_e = (wid + nw * k) * _C
            pltpu.sync_copy(adj_hbm.at[0, pl.ds(base_e, _C)], src_v)
            pltpu.sync_copy(adj_hbm.at[1, pl.ds(base_e, _C)], dst_v)
            pltpu.async_copy(x_hbm.at[src_v], rows_v, sem).wait()
            pltpu.sync_copy(rows_v, accum.at[dst_v], add=True)
            return carry

        lax.fori_loop(0, nk, body, 0)
        plsc.subcore_barrier()
        pltpu.sync_copy(accum.at[pl.ds(base_row, rps)],
                        out_hbm.at[c, pl.ds(base_row, rps)])

    return sc_agg


def _make_tc_head(N, H, O, BR):
    def tc_body(p_ref, w_ref, b_ref, mw_ref, mb_ref, out_ref):
        p = p_ref[0] + p_ref[1]
        h = jnp.dot(p, w_ref[...], preferred_element_type=jnp.float32)
        h = jnp.maximum(h + b_ref[...], 0.0)
        out_ref[...] = (
            jnp.dot(h, mw_ref[...], preferred_element_type=jnp.float32)
            + mb_ref[...])

    return pl.pallas_call(
        tc_body,
        grid=(N // BR,),
        in_specs=[
            pl.BlockSpec((2, BR, H), lambda i: (0, i, 0)),
            pl.BlockSpec((H, H), lambda i: (0, 0)),
            pl.BlockSpec((1, H), lambda i: (0, 0)),
            pl.BlockSpec((H, O), lambda i: (0, 0)),
            pl.BlockSpec((1, O), lambda i: (0, 0)),
        ],
        out_specs=pl.BlockSpec((BR, O), lambda i: (i, 0)),
        out_shape=jax.ShapeDtypeStruct((N, O), jnp.float32),
    )


def kernel(x, adj, W, b, mlp_W, mlp_b):
    N, H = x.shape
    E = adj.shape[1]
    nclass = mlp_W.shape[0]
    blk = _NS * _C
    NP = ((N + blk - 1) // blk) * blk
    partials = _make_sc_agg(NP, H, E)(x, adj)

    O = 128
    mwp = jnp.pad(mlp_W.T, ((0, 0), (0, O - nclass)))
    mbp = jnp.pad(mlp_b, (0, O - nclass)).reshape(1, O)
    outp = _make_tc_head(NP, H, O, BR=2048)(
        partials, W, b.reshape(1, H), mwp, mbp)
    return outp[:N, :nclass]


# speedup vs baseline: 11.7629x; 1.6682x over previous
import functools

import jax
import jax.numpy as jnp
from jax import lax
from jax.experimental import pallas as pl
from jax.experimental.pallas import tpu as pltpu
from jax.experimental.pallas import tpu_sc as plsc

_C = 128
_NC = 2
_NS = 16


_K = 8


def _make_sc_agg(NP, H, E):
    nchunk = E // _C
    nw = _NC * _NS
    rps = NP // _NS
    q, rem = divmod(nchunk, nw)
    ngroups = (q + (1 if rem else 0) + _K - 1) // _K
    ngroups += ngroups % 2
    mesh = plsc.VectorSubcoreMesh(
        core_axis_name="c", subcore_axis_name="s",
        num_cores=_NC, num_subcores=_NS)

    @functools.partial(
        pl.kernel,
        out_type=jax.ShapeDtypeStruct((_NC, NP, H), jnp.float32),
        mesh=mesh,
        scratch_types=[
            pltpu.VMEM((_K, 2, _C), jnp.int32),
            pltpu.VMEM((_K, 2, _C), jnp.int32),
            pltpu.VMEM((_C, H), jnp.float32),
            pltpu.VMEM((_C, H), jnp.float32),
            pltpu.VMEM_SHARED((NP, H), jnp.float32),
            pltpu.SemaphoreType.DMA,
            pltpu.SemaphoreType.DMA,
            pltpu.SemaphoreType.DMA,
        ],
    )
    def sc_agg(x_hbm, adj_hbm, out_hbm, idx_a, idx_b, rows_a, rows_b,
               accum, sg_a, sg_b, si):
        c = lax.axis_index("c")
        s = lax.axis_index("s")
        wid = s * _NC + c
        base_row = s * rps
        idx = (idx_a, idx_b)
        rows = (rows_a, rows_b)
        sg = (sg_a, sg_b)

        def zbody(i, carry):
            for j in range(H // 16):
                rows_a[i, pl.ds(j * 16, 16)] = jnp.zeros((16,), jnp.float32)
            return carry

        lax.fori_loop(0, _C, zbody, 0)
        for k in range(rps // _C):
            pltpu.sync_copy(rows_a, accum.at[pl.ds(base_row + k * _C, _C)])
        plsc.subcore_barrier()

        start = wid * q + jnp.minimum(wid, rem)
        end = start + q + jnp.where(wid < rem, 1, 0)

        pltpu.sync_copy(adj_hbm.at[pl.ds(start, _K)], idx_a)
        pltpu.async_copy(x_hbm.at[idx_a.at[0, 0]], rows_a, sg_a)

        def group_body(g2, carry):
            for gg in range(2):
                g = 2 * g2 + gg
                c0 = start + g * _K
                ib, nb = idx[gg], idx[1 - gg]
                pltpu.async_copy(adj_hbm.at[pl.ds(c0 + _K, _K)], nb, si)
                for j in range(_K):
                    ck = c0 + j
                    b = j % 2
                    nxt = ck + 1

                    @pl.when(ck < end)
                    def _wait():
                        pltpu.make_async_copy(
                            x_hbm.at[ib.at[j, 0]], rows[b], sg[b]).wait()

                    if j < _K - 1:
                        @pl.when(nxt < end)
                        def _issue():
                            pltpu.async_copy(
                                x_hbm.at[ib.at[j + 1, 0]], rows[1 - b],
                                sg[1 - b])
                    else:
                        pltpu.make_async_copy(
                            adj_hbm.at[pl.ds(c0 + _K, _K)], nb, si).wait()

                        @pl.when(nxt < end)
                        def _issue():
                            pltpu.async_copy(
                                x_hbm.at[nb.at[0, 0]], rows[1 - b], sg[1 - b])

                    @pl.when(ck < end)
                    def _scatter():
                        pltpu.sync_copy(rows[b], accum.at[ib.at[j, 1]],
                                        add=True)
            return carry

        lax.fori_loop(0, ngroups // 2, group_body, 0)
        plsc.subcore_barrier()
        pltpu.sync_copy(accum.at[pl.ds(base_row, rps)],
                        out_hbm.at[c, pl.ds(base_row, rps)])

    return sc_agg


def _make_tc_head(N, H, O, BR):
    def tc_body(p_ref, w_ref, b_ref, mw_ref, mb_ref, out_ref):
        p = p_ref[0] + p_ref[1]
        h = jnp.dot(p, w_ref[...], preferred_element_type=jnp.float32)
        h = jnp.maximum(h + b_ref[...], 0.0)
        out_ref[...] = (
            jnp.dot(h, mw_ref[...], preferred_element_type=jnp.float32)
            + mb_ref[...])

    return pl.pallas_call(
        tc_body,
        grid=(N // BR,),
        in_specs=[
            pl.BlockSpec((2, BR, H), lambda i: (0, i, 0)),
            pl.BlockSpec((H, H), lambda i: (0, 0)),
            pl.BlockSpec((1, H), lambda i: (0, 0)),
            pl.BlockSpec((H, O), lambda i: (0, 0)),
            pl.BlockSpec((1, O), lambda i: (0, 0)),
        ],
        out_specs=pl.BlockSpec((BR, O), lambda i: (i, 0)),
        out_shape=jax.ShapeDtypeStruct((N, O), jnp.float32),
    )


def kernel(x, adj, W, b, mlp_W, mlp_b):
    N, H = x.shape
    E = adj.shape[1]
    nclass = mlp_W.shape[0]
    blk = _NS * _C
    NP = ((N + blk - 1) // blk) * blk
    nchunk = E // _C
    nw = _NC * _NS
    q, rem = divmod(nchunk, nw)
    ngroups = (q + (1 if rem else 0) + _K - 1) // _K
    ngroups += ngroups % 2
    nchunk_pad = (nw - 1) * q + rem + (ngroups + 1) * _K
    adj4 = jnp.stack(
        [adj[0].reshape(nchunk, _C), adj[1].reshape(nchunk, _C)], axis=1)
    adj4 = jnp.pad(adj4, ((0, nchunk_pad - nchunk), (0, 0), (0, 0)))
    partials = _make_sc_agg(NP, H, E)(x, adj4)

    O = 128
    mwp = jnp.pad(mlp_W.T, ((0, 0), (0, O - nclass)))
    mbp = jnp.pad(mlp_b, (0, O - nclass)).reshape(1, O)
    outp = _make_tc_head(NP, H, O, BR=2048)(
        partials, W, b.reshape(1, H), mwp, mbp)
    return outp[:N, :nclass]
